# Initial kernel scaffold; baseline (speedup 1.0000x reference)
import jax
import jax.numpy as jnp
from jax.experimental import pallas as pl


def kernel(edge_index, node_sh):
    raise NotImplementedError("write your pallas kernel here")


# speedup vs baseline: 10.7286x; 10.7286x over previous
import functools

import jax
import jax.numpy as jnp
from jax import lax
from jax.experimental import pallas as pl
from jax.experimental.pallas import tpu as pltpu
from jax.experimental.pallas import tpu_sc as plsc

LMAX = 3
SH_DIM = (LMAX + 1) ** 2
N_NODES_C = 100000
N_EDGES_C = 3200000

NC, NS, L = 2, 16, 16
NW = NC * NS

PER_TILE = N_EDGES_C // NW
B = 800
CHUNKS = PER_TILE // B
GROUPS = B // L

_F2L = [0] + [1] * 3 + [2] * 5 + [3] * 7


def _sh_msg_body(edge_hbm, node_hbm, out_hbm,
                 row_idx, col_idx, r_rows, c_rows, out_buf, sem):
    wid = lax.axis_index("s") * NC + lax.axis_index("c")
    tile_base = wid * PER_TILE

    lane = lax.iota(jnp.int32, L)

    def chunk_body(k, carry):
        s = tile_base + k * B
        pltpu.sync_copy(edge_hbm.at[pl.ds(s, B)], row_idx)
        pltpu.sync_copy(edge_hbm.at[pl.ds(N_EDGES_C + s, B)], col_idx)
        cp_r = pltpu.async_copy(node_hbm.at[row_idx], r_rows, sem)
        cp_c = pltpu.async_copy(node_hbm.at[col_idx], c_rows, sem)
        cp_r.wait()
        cp_c.wait()

        def group_body(g, gcarry):
            eidx = g * L + lane
            accs = [None] * (LMAX + 1)
            for f in range(SH_DIM):
                fv = jnp.full((L,), f, jnp.int32)
                rf = plsc.load_gather(r_rows, [eidx, fv])
                cf = plsc.load_gather(c_rows, [eidx, fv])
                p = rf * cf
                l = _F2L[f]
                accs[l] = p if accs[l] is None else accs[l] + p
            for l in range(LMAX + 1):
                lv = jnp.full((L,), l, jnp.int32)
                plsc.store_scatter(out_buf, [eidx, lv], accs[l])
            return gcarry

        lax.fori_loop(0, GROUPS, group_body, 0)
        pltpu.sync_copy(out_buf, out_hbm.at[pl.ds(s, B)])
        return carry

    lax.fori_loop(0, CHUNKS, chunk_body, 0)


@jax.jit
def _sh_msg(edge_index, node_sh):
    mesh = plsc.VectorSubcoreMesh(
        core_axis_name="c", subcore_axis_name="s",
        num_cores=NC, num_subcores=NS)
    return pl.kernel(
        _sh_msg_body,
        out_type=jax.ShapeDtypeStruct((N_EDGES_C, LMAX + 1), jnp.float32),
        mesh=mesh,
        scratch_types=[
            pltpu.VMEM((B,), jnp.int32),
            pltpu.VMEM((B,), jnp.int32),
            pltpu.VMEM((B, SH_DIM), jnp.float32),
            pltpu.VMEM((B, SH_DIM), jnp.float32),
            pltpu.VMEM((B, LMAX + 1), jnp.float32),
            pltpu.SemaphoreType.DMA,
        ],
        compiler_params=pltpu.CompilerParams(
            needs_layout_passes=False, use_tc_tiling_on_sc=False),
    )(edge_index, node_sh)


def kernel(edge_index, node_sh):
    assert edge_index.shape == (2, N_EDGES_C)
    assert node_sh.shape == (N_NODES_C, SH_DIM)
    return _sh_msg(edge_index.reshape(2 * N_EDGES_C), node_sh)
